# baseline (speedup 1.0000x reference)
import functools

import jax
import jax.numpy as jnp
from jax import lax
from jax.experimental import pallas as pl
from jax.experimental.pallas import tpu as pltpu
from jax.experimental.pallas import tpu_sc as plsc

_NUM_TEMPLATES = 1000
_NUM_ROLES = 64
_ROLE_DIM = 128
_B = 4096
_N = 50

_NW = 32
_CHUNK = _B * _N // _NW
_G = 80
_NG = _CHUNK // _G
_NBUF = 10
_LEAD = 6


def _sc_body(tid_hbm, role_hbm, tbl_hbm, out_hbm,
             tid_v, role_v, idx_v, bufs, gsems, ssems):
    cid = lax.axis_index("c")
    sid = lax.axis_index("s")
    wid = cid * 16 + sid
    base = wid * _CHUNK

    pltpu.sync_copy(tid_hbm, tid_v)
    pltpu.sync_copy(role_hbm.at[pl.ds(base, _CHUNK)], role_v)

    def idx_body(k, _):
        p0 = k * 16
        i0 = (base + p0) & (_B - 1)
        tv = tid_v[pl.ds(i0, 16)]
        rv = role_v[pl.ds(p0, 16)]
        idx_v[k // (_G // 16), pl.ds((k % (_G // 16)) * 16, 16)] = tv * _NUM_ROLES + rv
        return 0

    lax.fori_loop(0, _CHUNK // 16, idx_body, 0)

    def fire_gather(g, b):
        pltpu.async_copy(tbl_hbm.at[idx_v.at[g]], bufs[b], gsems[b])

    def wait_gather(g, b):
        pltpu.make_async_copy(
            tbl_hbm.at[idx_v.at[g]], bufs[b], gsems[b]).wait()

    def fire_scatter(g, b):
        pltpu.async_copy(
            bufs[b], out_hbm.at[pl.ds(base + g * _G, _G)], ssems[b])

    def wait_scatter(b):
        pltpu.make_async_copy(
            bufs[b], out_hbm.at[pl.ds(base, _G)], ssems[b]).wait()

    for g in range(_LEAD):
        fire_gather(g, g)

    def g_body(i, _):
        for b in range(_NBUF):
            g = i * _NBUF + b
            bf = (b + _LEAD) % _NBUF
            nf = g + _LEAD

            @pl.when(g < _NG)
            def _():
                @pl.when(jnp.logical_and(nf < _NG, nf >= _NBUF))
                def _():
                    wait_scatter(bf)

                @pl.when(nf < _NG)
                def _():
                    fire_gather(nf, bf)

                wait_gather(g, b)
                fire_scatter(g, b)
        return 0

    lax.fori_loop(0, (_NG + _NBUF - 1) // _NBUF, g_body, 0)

    for b in range(_NBUF):
        wait_scatter(b)


@jax.jit
def _lookup(tid, role_t_flat, tbl_flat):
    mesh = plsc.VectorSubcoreMesh(core_axis_name="c", subcore_axis_name="s")
    kfn = functools.partial(
        pl.kernel,
        mesh=mesh,
        compiler_params=pltpu.CompilerParams(needs_layout_passes=False),
        out_type=jax.ShapeDtypeStruct((_N * _B, _ROLE_DIM), jnp.float32),
        scratch_types=[
            pltpu.VMEM((_B,), jnp.int32),
            pltpu.VMEM((_CHUNK,), jnp.int32),
            pltpu.VMEM((_NG, _G), jnp.int32),
            [pltpu.VMEM((_G, _ROLE_DIM), jnp.float32) for _ in range(_NBUF)],
            [pltpu.SemaphoreType.DMA for _ in range(_NBUF)],
            [pltpu.SemaphoreType.DMA for _ in range(_NBUF)],
        ],
    )(_sc_body)
    return kfn(tid, role_t_flat, tbl_flat)


def kernel(template_id_int, atom_role, tables):
    tid = template_id_int.astype(jnp.int32)
    role_t_flat = atom_role.astype(jnp.int32).T.reshape(_N * _B)
    tbl_flat = tables.reshape(_NUM_TEMPLATES * _NUM_ROLES, _ROLE_DIM)
    out = _lookup(tid, role_t_flat, tbl_flat)
    return out.reshape(_N, _B, _ROLE_DIM).transpose(1, 0, 2)


# speedup vs baseline: 1.0055x; 1.0055x over previous
import functools

import jax
import jax.numpy as jnp
from jax import lax
from jax.experimental import pallas as pl
from jax.experimental.pallas import tpu as pltpu
from jax.experimental.pallas import tpu_sc as plsc

_NUM_TEMPLATES = 1000
_NUM_ROLES = 64
_ROLE_DIM = 128
_B = 4096
_N = 50

_NW = 32
_CHUNK = _B * _N // _NW
_G = 80
_NG = _CHUNK // _G
_NBUF = 6
_LEAD = 4


def _sc_body(tid_hbm, role_hbm, tbl_hbm, out_hbm,
             tid_v, role_v, idx_v, spm, bufs, gsems, ssems, s2sems):
    cid = lax.axis_index("c")
    sid = lax.axis_index("s")
    wid = cid * 16 + sid
    base = wid * _CHUNK

    pltpu.sync_copy(tid_hbm, tid_v)
    pltpu.sync_copy(role_hbm.at[pl.ds(base, _CHUNK)], role_v)

    def idx_body(k, _):
        p0 = k * 16
        i0 = (base + p0) & (_B - 1)
        tv = tid_v[pl.ds(i0, 16)]
        rv = role_v[pl.ds(p0, 16)]
        idx_v[k // (_G // 16), pl.ds((k % (_G // 16)) * 16, 16)] = tv * _NUM_ROLES + rv
        return 0

    lax.fori_loop(0, _CHUNK // 16, idx_body, 0)

    def fire_gather(g, b):
        pltpu.async_copy(tbl_hbm.at[idx_v.at[g]], bufs[b], gsems[b])

    def wait_gather(g, b):
        pltpu.make_async_copy(
            tbl_hbm.at[idx_v.at[g]], bufs[b], gsems[b]).wait()

    def fire_scatter(g, b):
        slot = b % 2

        @pl.when(g >= 2)
        def _():
            pltpu.make_async_copy(
                spm.at[sid, slot], out_hbm.at[pl.ds(base, _G)],
                s2sems[slot]).wait()

        pltpu.async_copy(bufs[b], spm.at[sid, slot], ssems[b])
        pltpu.make_async_copy(bufs[b], spm.at[sid, slot], ssems[b]).wait()
        pltpu.async_copy(
            spm.at[sid, slot], out_hbm.at[pl.ds(base + g * _G, _G)],
            s2sems[slot])

    for g in range(_LEAD):
        fire_gather(g, g)

    def g_body(i, _):
        for b in range(_NBUF):
            g = i * _NBUF + b
            bf = (b + _LEAD) % _NBUF
            nf = g + _LEAD

            @pl.when(g < _NG)
            def _():
                @pl.when(nf < _NG)
                def _():
                    fire_gather(nf, bf)

                wait_gather(g, b)
                fire_scatter(g, b)
        return 0

    lax.fori_loop(0, (_NG + _NBUF - 1) // _NBUF, g_body, 0)

    for slot in range(2):
        pltpu.make_async_copy(
            spm.at[sid, slot], out_hbm.at[pl.ds(base, _G)],
            s2sems[slot]).wait()


@jax.jit
def _lookup(tid, role_t_flat, tbl_flat):
    mesh = plsc.VectorSubcoreMesh(core_axis_name="c", subcore_axis_name="s")
    kfn = functools.partial(
        pl.kernel,
        mesh=mesh,
        compiler_params=pltpu.CompilerParams(needs_layout_passes=False),
        out_type=jax.ShapeDtypeStruct((_N * _B, _ROLE_DIM), jnp.float32),
        scratch_types=[
            pltpu.VMEM((_B,), jnp.int32),
            pltpu.VMEM((_CHUNK,), jnp.int32),
            pltpu.VMEM((_NG, _G), jnp.int32),
            pltpu.VMEM_SHARED((16, 2, _G, _ROLE_DIM), jnp.float32),
            [pltpu.VMEM((_G, _ROLE_DIM), jnp.float32) for _ in range(_NBUF)],
            [pltpu.SemaphoreType.DMA for _ in range(_NBUF)],
            [pltpu.SemaphoreType.DMA for _ in range(_NBUF)],
            [pltpu.SemaphoreType.DMA for _ in range(2)],
        ],
    )(_sc_body)
    return kfn(tid, role_t_flat, tbl_flat)


def kernel(template_id_int, atom_role, tables):
    tid = template_id_int.astype(jnp.int32)
    role_t_flat = atom_role.astype(jnp.int32).T.reshape(_N * _B)
    tbl_flat = tables.reshape(_NUM_TEMPLATES * _NUM_ROLES, _ROLE_DIM)
    out = _lookup(tid, role_t_flat, tbl_flat)
    return out.reshape(_N, _B, _ROLE_DIM).transpose(1, 0, 2)
